# baseline (speedup 1.0000x reference)
import functools

import jax
import jax.numpy as jnp
from jax import lax
from jax.experimental import pallas as pl
from jax.experimental.pallas import tpu as pltpu
from jax.experimental.pallas import tpu_sc as plsc

_N = 50000
_E = 1600000
_CIN = 8
_COUT = 16
_F = _CIN * 4
_FH = _F // 2

_NC = 2
_NS = 16
_IW = 128
_CH_ROWS = 16
_CH = _IW * _CH_ROWS
_EPAD = 12800 * _IW
_EPT = _EPAD // _NS
_NCHUNK = _EPT // _CH
_NIO = 10
_RPT = _N // _NIO


@functools.lru_cache(maxsize=None)
def _make_sc_agg():
    mesh = plsc.VectorSubcoreMesh(core_axis_name="c", subcore_axis_name="s")

    @functools.partial(
        pl.kernel,
        mesh=mesh,
        compiler_params=pltpu.CompilerParams(use_tc_tiling_on_sc=False),
        out_type=jax.ShapeDtypeStruct((_NC, _N, _FH), jnp.float32),
        scratch_types=[
            pltpu.VMEM((2, _CH), jnp.int32),
            pltpu.VMEM((2, _CH), jnp.int32),
            pltpu.VMEM((2, _CH_ROWS, _IW), jnp.int32),
            pltpu.VMEM((2, _CH, _FH), jnp.float32),
            pltpu.VMEM_SHARED((_N, _FH), jnp.float32),
            pltpu.SemaphoreType.DMA,
            pltpu.SemaphoreType.DMA,
            pltpu.SemaphoreType.DMA,
        ],
    )
    def _sc_agg(x_hbm, src_hbm, dst_hbm, zero_hbm, out_hbm,
                src_v, gid_v, dst_v, rows_v, agg_sh, semg, semi, sems):
        c = lax.axis_index("c")
        s = lax.axis_index("s")

        @pl.when(s < _NIO)
        def _init():
            pltpu.sync_copy(zero_hbm, agg_sh.at[pl.ds(s * _RPT, _RPT)])
        plsc.subcore_barrier()

        e0 = s * _EPT
        drow0 = e0 // _IW

        def load_idx(k, b):
            pltpu.async_copy(src_hbm.at[pl.ds(e0 + k * _CH, _CH)],
                             src_v.at[b], semi)
            pltpu.async_copy(dst_hbm.at[pl.ds(drow0 + k * _CH_ROWS, _CH_ROWS)],
                             dst_v.at[b], semi)

        def wait_idx(b):
            pltpu.make_async_copy(src_hbm.at[pl.ds(0, _CH)],
                                  src_v.at[b], semi).wait()
            pltpu.make_async_copy(dst_hbm.at[pl.ds(0, _CH_ROWS)],
                                  dst_v.at[b], semi).wait()

        def make_gids(b):
            @pl.loop(0, _CH // 16)
            def _t(i):
                t = src_v[b, pl.ds(i * 16, 16)]
                gid_v[b, pl.ds(i * 16, 16)] = t + t + c

        def fire_gathers(b):
            for j in range(_CH_ROWS):
                pltpu.async_copy(x_hbm.at[gid_v.at[b, pl.ds(j * _IW, _IW)]],
                                 rows_v.at[b, pl.ds(j * _IW, _IW)], semg)

        def wait_gathers(b):
            for j in range(_CH_ROWS):
                pltpu.make_async_copy(
                    x_hbm.at[gid_v.at[b, pl.ds(j * _IW, _IW)]],
                    rows_v.at[b, pl.ds(j * _IW, _IW)], semg).wait()

        def scatter_chunk(b):
            for j in range(_CH_ROWS):
                pltpu.async_copy(rows_v.at[b, pl.ds(j * _IW, _IW)],
                                 agg_sh.at[dst_v.at[b, j]], sems, add=True)
            for j in range(_CH_ROWS):
                pltpu.make_async_copy(rows_v.at[b, pl.ds(j * _IW, _IW)],
                                      agg_sh.at[dst_v.at[b, j]],
                                      sems).wait()

        load_idx(0, 0)
        wait_idx(0)
        make_gids(0)
        fire_gathers(0)
        load_idx(1, 1)

        @pl.loop(0, _NCHUNK // 2)
        def _pair(m):
            for b in (0, 1):
                k = m * 2 + b

                @pl.when(k + 1 < _NCHUNK)
                def _prefetch():
                    wait_idx(1 - b)
                    make_gids(1 - b)
                    fire_gathers(1 - b)

                wait_gathers(b)
                scatter_chunk(b)

                @pl.when(k + 2 < _NCHUNK)
                def _nextidx():
                    load_idx(k + 2, b)

        plsc.subcore_barrier()

        @pl.when(s < _NIO)
        def _writeback():
            pltpu.sync_copy(agg_sh.at[pl.ds(s * _RPT, _RPT)],
                            out_hbm.at[c, pl.ds(s * _RPT, _RPT)])

    return _sc_agg


_NB = 512


def _routing_body(agg_ref, quat_ref, ab_ref, out_ref):
    eps = 1e-8
    a = jnp.transpose(agg_ref[...])
    a = a.reshape(_CIN, 4, _NB)
    pw, px, py, pz = a[:, 0], a[:, 1], a[:, 2], a[:, 3]
    inv = 1.0 / (jnp.sqrt(pw * pw + px * px + py * py + pz * pz) + eps)
    pw, px, py, pz = pw * inv, px * inv, py * inv, pz * inv

    qw = quat_ref[0:16, :][:, :, None]
    qx = quat_ref[16:32, :][:, :, None]
    qy = quat_ref[32:48, :][:, :, None]
    qz = quat_ref[48:64, :][:, :, None]
    bw, bx, by, bz = pw[None], px[None], py[None], pz[None]
    vw = qw * bw - qx * bx - qy * by - qz * bz
    vx = qw * bx + qx * bw + qy * bz - qz * by
    vy = qw * by - qx * bz + qy * bw + qz * bx
    vz = qw * bz + qx * by - qy * bx + qz * bw

    def pose_and_agree(sw, sx, sy, sz):
        inv = 1.0 / (jnp.sqrt(sw * sw + sx * sx + sy * sy + sz * sz) + eps)
        ow, ox, oy, oz = sw * inv, sx * inv, sy * inv, sz * inv
        agree = (vw * ow[:, None] + vx * ox[:, None]
                 + vy * oy[:, None] + vz * oz[:, None])
        return ow, ox, oy, oz, agree

    sixteenth = jnp.float32(1.0 / _COUT)
    ow, ox, oy, oz, agree = pose_and_agree(
        vw.sum(axis=1) * sixteenth, vx.sum(axis=1) * sixteenth,
        vy.sum(axis=1) * sixteenth, vz.sum(axis=1) * sixteenth)
    b = agree

    for it in range(2):
        e = jnp.exp(b)
        cz = (1.0 / e.sum(axis=0))[None]
        c = e * cz
        ow, ox, oy, oz, agree = pose_and_agree(
            (c * vw).sum(axis=1), (c * vx).sum(axis=1),
            (c * vy).sum(axis=1), (c * vz).sum(axis=1))
        if it == 0:
            b = b + agree

    al = ab_ref[:, 0:1]
    be = ab_ref[:, 1:2]
    act = jax.nn.sigmoid(al * (agree.sum(axis=1) * jnp.float32(0.125)) + be)
    out = jnp.stack([ow * act, ox * act, oy * act, oz * act], axis=1)
    out_ref[...] = jnp.transpose(out.reshape(64, _NB))


def _routing_call(agg_nm, quat_n, ab):
    grid = (pl.cdiv(_N, _NB),)
    return pl.pallas_call(
        _routing_body,
        grid=grid,
        in_specs=[
            pl.BlockSpec((_NB, _F), lambda i: (i, 0)),
            pl.BlockSpec((64, 8), lambda i: (0, 0)),
            pl.BlockSpec((16, 2), lambda i: (0, 0)),
        ],
        out_specs=pl.BlockSpec((_NB, 64), lambda i: (i, 0)),
        out_shape=jax.ShapeDtypeStruct((_N, 64), jnp.float32),
    )(agg_nm, quat_n, ab)


def kernel(x, edge_index, quaternions, alpha, beta):
    x2r = jnp.concatenate(
        [x.reshape(2 * _N, _FH), jnp.zeros((2, _FH), jnp.float32)])
    npad = _EPAD - _E
    src = jnp.concatenate([edge_index[0], jnp.full((npad,), _N, jnp.int32)])
    dst2 = jnp.concatenate(
        [edge_index[1], jnp.arange(npad, dtype=jnp.int32)]
    ).reshape(_EPAD // _IW, _IW)
    zero = jnp.zeros((_RPT, _FH), jnp.float32)
    aggs = _make_sc_agg()(x2r, src, dst2, zero)
    agg_nm = aggs.transpose(1, 0, 2).reshape(_N, _F)
    qnorm = jnp.linalg.norm(quaternions, axis=-1, keepdims=True)
    quat_n = (quaternions / (qnorm + 1e-8)).transpose(2, 0, 1).reshape(64, _CIN)
    ab = jnp.stack([alpha, beta], axis=1)
    out2d = _routing_call(agg_nm, quat_n, ab)
    return out2d.reshape(_N, _COUT, 4)


# speedup vs baseline: 1.3648x; 1.3648x over previous
import functools

import jax
import jax.numpy as jnp
from jax import lax
from jax.experimental import pallas as pl
from jax.experimental.pallas import tpu as pltpu
from jax.experimental.pallas import tpu_sc as plsc

_N = 50000
_E = 1600000
_CIN = 8
_COUT = 16
_F = _CIN * 4
_FH = _F // 2

_NC = 2
_NS = 16
_IW = 125
_CH_ROWS = 16
_CH = _IW * _CH_ROWS
_EPT = _E // _NS
_NCHUNK = _EPT // _CH
_NIO = 10
_RPT = _N // _NIO


@functools.lru_cache(maxsize=None)
def _make_sc_agg():
    mesh = plsc.VectorSubcoreMesh(core_axis_name="c", subcore_axis_name="s")

    @functools.partial(
        pl.kernel,
        mesh=mesh,
        compiler_params=pltpu.CompilerParams(use_tc_tiling_on_sc=False),
        out_type=jax.ShapeDtypeStruct((_N, _F), jnp.float32),
        scratch_types=[
            pltpu.VMEM((2, _CH_ROWS, _IW), jnp.int32),
            pltpu.VMEM((2, _CH_ROWS, _IW), jnp.int32),
            pltpu.VMEM((2, _CH, _FH), jnp.float32),
            pltpu.VMEM_SHARED((_N, _FH), jnp.float32),
            pltpu.SemaphoreType.DMA,
            pltpu.SemaphoreType.DMA,
            pltpu.SemaphoreType.DMA,
        ],
    )
    def _sc_agg(x_hbm, gid_hbm, dst_hbm, zero_hbm, out_hbm,
                gid_v, dst_v, rows_v, agg_sh, semg, semi, sems):
        c = lax.axis_index("c")
        s = lax.axis_index("s")

        @pl.when(s < _NIO)
        def _init():
            pltpu.sync_copy(zero_hbm, agg_sh.at[pl.ds(s * _RPT, _RPT)])
        plsc.subcore_barrier()

        row0 = s * (_EPT // _IW)

        def load_idx(k, b):
            base = row0 + k * _CH_ROWS
            pltpu.async_copy(gid_hbm.at[c, pl.ds(base, _CH_ROWS)],
                             gid_v.at[b], semi)
            pltpu.async_copy(dst_hbm.at[pl.ds(base, _CH_ROWS)],
                             dst_v.at[b], semi)

        def wait_idx(b):
            pltpu.make_async_copy(gid_hbm.at[0, pl.ds(0, _CH_ROWS)],
                                  gid_v.at[b], semi).wait()
            pltpu.make_async_copy(dst_hbm.at[pl.ds(0, _CH_ROWS)],
                                  dst_v.at[b], semi).wait()

        def fire_gathers(b):
            for j in range(_CH_ROWS):
                pltpu.async_copy(x_hbm.at[gid_v.at[b, j]],
                                 rows_v.at[b, pl.ds(j * _IW, _IW)], semg)

        def wait_gathers(b):
            for j in range(_CH_ROWS):
                pltpu.make_async_copy(
                    x_hbm.at[gid_v.at[b, j]],
                    rows_v.at[b, pl.ds(j * _IW, _IW)], semg).wait()

        def scatter_chunk(b):
            for j in range(_CH_ROWS):
                pltpu.async_copy(rows_v.at[b, pl.ds(j * _IW, _IW)],
                                 agg_sh.at[dst_v.at[b, j]], sems, add=True)
            for j in range(_CH_ROWS):
                pltpu.make_async_copy(rows_v.at[b, pl.ds(j * _IW, _IW)],
                                      agg_sh.at[dst_v.at[b, j]],
                                      sems).wait()

        load_idx(0, 0)
        wait_idx(0)
        fire_gathers(0)
        load_idx(1, 1)

        @pl.loop(0, _NCHUNK // 2)
        def _pair(m):
            for b in (0, 1):
                k = m * 2 + b

                @pl.when(k + 1 < _NCHUNK)
                def _prefetch():
                    wait_idx(1 - b)
                    fire_gathers(1 - b)

                wait_gathers(b)
                scatter_chunk(b)

                @pl.when(k + 2 < _NCHUNK)
                def _nextidx():
                    load_idx(k + 2, b)

        plsc.subcore_barrier()

        @pl.when(s < _NIO)
        def _writeback():
            pltpu.sync_copy(agg_sh.at[pl.ds(s * _RPT, _RPT)],
                            out_hbm.at[pl.ds(s * _RPT, _RPT),
                                       pl.ds(c * _FH, _FH)])

    return _sc_agg


_NB = 512


def _routing_body(agg_ref, quat_ref, ab_ref, out_ref):
    eps = 1e-8
    a = jnp.transpose(agg_ref[...])
    a = a.reshape(_CIN, 4, _NB)
    pw, px, py, pz = a[:, 0], a[:, 1], a[:, 2], a[:, 3]
    inv = 1.0 / (jnp.sqrt(pw * pw + px * px + py * py + pz * pz) + eps)
    pw, px, py, pz = pw * inv, px * inv, py * inv, pz * inv

    qw = quat_ref[0:16, :][:, :, None]
    qx = quat_ref[16:32, :][:, :, None]
    qy = quat_ref[32:48, :][:, :, None]
    qz = quat_ref[48:64, :][:, :, None]
    bw, bx, by, bz = pw[None], px[None], py[None], pz[None]
    vw = qw * bw - qx * bx - qy * by - qz * bz
    vx = qw * bx + qx * bw + qy * bz - qz * by
    vy = qw * by - qx * bz + qy * bw + qz * bx
    vz = qw * bz + qx * by - qy * bx + qz * bw

    def pose_and_agree(sw, sx, sy, sz):
        inv = 1.0 / (jnp.sqrt(sw * sw + sx * sx + sy * sy + sz * sz) + eps)
        ow, ox, oy, oz = sw * inv, sx * inv, sy * inv, sz * inv
        agree = (vw * ow[:, None] + vx * ox[:, None]
                 + vy * oy[:, None] + vz * oz[:, None])
        return ow, ox, oy, oz, agree

    sixteenth = jnp.float32(1.0 / _COUT)
    ow, ox, oy, oz, agree = pose_and_agree(
        vw.sum(axis=1) * sixteenth, vx.sum(axis=1) * sixteenth,
        vy.sum(axis=1) * sixteenth, vz.sum(axis=1) * sixteenth)
    b = agree

    for it in range(2):
        e = jnp.exp(b)
        cz = (1.0 / e.sum(axis=0))[None]
        c = e * cz
        ow, ox, oy, oz, agree = pose_and_agree(
            (c * vw).sum(axis=1), (c * vx).sum(axis=1),
            (c * vy).sum(axis=1), (c * vz).sum(axis=1))
        if it == 0:
            b = b + agree

    al = ab_ref[:, 0:1]
    be = ab_ref[:, 1:2]
    act = jax.nn.sigmoid(al * (agree.sum(axis=1) * jnp.float32(0.125)) + be)
    out = jnp.stack([ow * act, ox * act, oy * act, oz * act], axis=1)
    out_ref[...] = jnp.transpose(out.reshape(64, _NB))


def _routing_call(agg_nm, quat_n, ab):
    grid = (pl.cdiv(_N, _NB),)
    return pl.pallas_call(
        _routing_body,
        grid=grid,
        in_specs=[
            pl.BlockSpec((_NB, _F), lambda i: (i, 0)),
            pl.BlockSpec((64, 8), lambda i: (0, 0)),
            pl.BlockSpec((16, 2), lambda i: (0, 0)),
        ],
        out_specs=pl.BlockSpec((_NB, 64), lambda i: (i, 0)),
        out_shape=jax.ShapeDtypeStruct((_N, 64), jnp.float32),
    )(agg_nm, quat_n, ab)


def kernel(x, edge_index, quaternions, alpha, beta):
    x2r = x.reshape(2 * _N, _FH)
    src = edge_index[0]
    gids = jnp.stack([src * 2, src * 2 + 1]).reshape(2, _E // _IW, _IW)
    dst2 = edge_index[1].reshape(_E // _IW, _IW)
    zero = jnp.zeros((_RPT, _FH), jnp.float32)
    agg_nm = _make_sc_agg()(x2r, gids, dst2, zero)
    qnorm = jnp.linalg.norm(quaternions, axis=-1, keepdims=True)
    quat_n = (quaternions / (qnorm + 1e-8)).transpose(2, 0, 1).reshape(64, _CIN)
    ab = jnp.stack([alpha, beta], axis=1)
    out2d = _routing_call(agg_nm, quat_n, ab)
    return out2d.reshape(_N, _COUT, 4)


# speedup vs baseline: 2.2499x; 1.6485x over previous
import functools

import jax
import jax.numpy as jnp
from jax import lax
from jax.experimental import pallas as pl
from jax.experimental.pallas import tpu as pltpu
from jax.experimental.pallas import tpu_sc as plsc

_N = 50000
_E = 1600000
_CIN = 8
_COUT = 16
_F = _CIN * 4
_FH = _F // 2

_NC = 2
_NS = 16
_IW = 125
_CH_ROWS = 16
_CH = _IW * _CH_ROWS
_EPT = _E // _NS
_NCHUNK = _EPT // _CH
_NIO = 10
_RPT = _N // _NIO


@functools.lru_cache(maxsize=None)
def _make_sc_agg():
    mesh = plsc.VectorSubcoreMesh(core_axis_name="c", subcore_axis_name="s")

    @functools.partial(
        pl.kernel,
        mesh=mesh,
        compiler_params=pltpu.CompilerParams(use_tc_tiling_on_sc=False),
        out_type=jax.ShapeDtypeStruct((_N, _F), jnp.float32),
        scratch_types=[
            pltpu.VMEM((2, _CH_ROWS, _IW), jnp.int32),
            pltpu.VMEM((2, _CH_ROWS, _IW), jnp.int32),
            pltpu.VMEM((2, _CH, _FH), jnp.float32),
            pltpu.VMEM_SHARED((_N, _FH), jnp.float32),
            pltpu.SemaphoreType.DMA,
            pltpu.SemaphoreType.DMA,
            pltpu.SemaphoreType.DMA,
        ],
    )
    def _sc_agg(xs_hbm, src_hbm, dst_hbm, zero_hbm, out_hbm,
                gid_v, dst_v, rows_v, agg_sh, semg, semi, sems):
        c = lax.axis_index("c")
        s = lax.axis_index("s")

        @pl.when(s < _NIO)
        def _init():
            pltpu.sync_copy(zero_hbm, agg_sh.at[pl.ds(s * _RPT, _RPT)])
        plsc.subcore_barrier()

        row0 = s * (_EPT // _IW)

        def load_idx(k, b):
            base = row0 + k * _CH_ROWS
            pltpu.async_copy(src_hbm.at[pl.ds(base, _CH_ROWS)],
                             gid_v.at[b], semi)
            pltpu.async_copy(dst_hbm.at[pl.ds(base, _CH_ROWS)],
                             dst_v.at[b], semi)

        def wait_idx(b):
            pltpu.make_async_copy(src_hbm.at[pl.ds(0, _CH_ROWS)],
                                  gid_v.at[b], semi).wait()
            pltpu.make_async_copy(dst_hbm.at[pl.ds(0, _CH_ROWS)],
                                  dst_v.at[b], semi).wait()

        def fire_gathers(b):
            for j in range(_CH_ROWS):
                pltpu.async_copy(xs_hbm.at[c].at[gid_v.at[b, j]],
                                 rows_v.at[b, pl.ds(j * _IW, _IW)], semg)

        def wait_gathers(b):
            for j in range(_CH_ROWS):
                pltpu.make_async_copy(
                    xs_hbm.at[c].at[gid_v.at[b, j]],
                    rows_v.at[b, pl.ds(j * _IW, _IW)], semg).wait()

        def scatter_chunk(b):
            for j in range(_CH_ROWS):
                pltpu.async_copy(rows_v.at[b, pl.ds(j * _IW, _IW)],
                                 agg_sh.at[dst_v.at[b, j]], sems, add=True)
            for j in range(_CH_ROWS):
                pltpu.make_async_copy(rows_v.at[b, pl.ds(j * _IW, _IW)],
                                      agg_sh.at[dst_v.at[b, j]],
                                      sems).wait()

        load_idx(0, 0)
        wait_idx(0)
        fire_gathers(0)
        load_idx(1, 1)

        @pl.loop(0, _NCHUNK // 2)
        def _pair(m):
            for b in (0, 1):
                k = m * 2 + b

                @pl.when(k + 1 < _NCHUNK)
                def _prefetch():
                    wait_idx(1 - b)
                    fire_gathers(1 - b)

                wait_gathers(b)
                scatter_chunk(b)

                @pl.when(k + 2 < _NCHUNK)
                def _nextidx():
                    load_idx(k + 2, b)

        plsc.subcore_barrier()

        @pl.when(s < _NIO)
        def _writeback():
            pltpu.sync_copy(agg_sh.at[pl.ds(s * _RPT, _RPT)],
                            out_hbm.at[pl.ds(s * _RPT, _RPT),
                                       pl.ds(c * _FH, _FH)])

    return _sc_agg


_NB = 512


def _routing_body(agg_ref, quat_ref, ab_ref, out_ref):
    eps = 1e-8
    a = jnp.transpose(agg_ref[...])
    a = a.reshape(_CIN, 4, _NB)
    pw, px, py, pz = a[:, 0], a[:, 1], a[:, 2], a[:, 3]
    inv = 1.0 / (jnp.sqrt(pw * pw + px * px + py * py + pz * pz) + eps)
    pw, px, py, pz = pw * inv, px * inv, py * inv, pz * inv

    qw = quat_ref[0:16, :][:, :, None]
    qx = quat_ref[16:32, :][:, :, None]
    qy = quat_ref[32:48, :][:, :, None]
    qz = quat_ref[48:64, :][:, :, None]
    bw, bx, by, bz = pw[None], px[None], py[None], pz[None]
    vw = qw * bw - qx * bx - qy * by - qz * bz
    vx = qw * bx + qx * bw + qy * bz - qz * by
    vy = qw * by - qx * bz + qy * bw + qz * bx
    vz = qw * bz + qx * by - qy * bx + qz * bw

    def pose_and_agree(sw, sx, sy, sz):
        inv = 1.0 / (jnp.sqrt(sw * sw + sx * sx + sy * sy + sz * sz) + eps)
        ow, ox, oy, oz = sw * inv, sx * inv, sy * inv, sz * inv
        agree = (vw * ow[:, None] + vx * ox[:, None]
                 + vy * oy[:, None] + vz * oz[:, None])
        return ow, ox, oy, oz, agree

    sixteenth = jnp.float32(1.0 / _COUT)
    ow, ox, oy, oz, agree = pose_and_agree(
        vw.sum(axis=1) * sixteenth, vx.sum(axis=1) * sixteenth,
        vy.sum(axis=1) * sixteenth, vz.sum(axis=1) * sixteenth)
    b = agree

    for it in range(2):
        e = jnp.exp(b)
        cz = (1.0 / e.sum(axis=0))[None]
        c = e * cz
        ow, ox, oy, oz, agree = pose_and_agree(
            (c * vw).sum(axis=1), (c * vx).sum(axis=1),
            (c * vy).sum(axis=1), (c * vz).sum(axis=1))
        if it == 0:
            b = b + agree

    al = ab_ref[:, 0:1]
    be = ab_ref[:, 1:2]
    act = jax.nn.sigmoid(al * (agree.sum(axis=1) * jnp.float32(0.125)) + be)
    out = jnp.stack([ow * act, ox * act, oy * act, oz * act], axis=1)
    out_ref[...] = jnp.transpose(out.reshape(64, _NB))


def _routing_call(agg_nm, quat_n, ab):
    grid = (pl.cdiv(_N, _NB),)
    return pl.pallas_call(
        _routing_body,
        grid=grid,
        in_specs=[
            pl.BlockSpec((_NB, _F), lambda i: (i, 0)),
            pl.BlockSpec((64, 8), lambda i: (0, 0)),
            pl.BlockSpec((16, 2), lambda i: (0, 0)),
        ],
        out_specs=pl.BlockSpec((_NB, 64), lambda i: (i, 0)),
        out_shape=jax.ShapeDtypeStruct((_N, 64), jnp.float32),
    )(agg_nm, quat_n, ab)


def kernel(x, edge_index, quaternions, alpha, beta):
    x2 = x.reshape(_N, _F)
    xs = jnp.stack([x2[:, :_FH], x2[:, _FH:]])
    src2 = edge_index[0].reshape(_E // _IW, _IW)
    dst2 = edge_index[1].reshape(_E // _IW, _IW)
    zero = jnp.zeros((_RPT, _FH), jnp.float32)
    agg_nm = _make_sc_agg()(xs, src2, dst2, zero)
    qnorm = jnp.linalg.norm(quaternions, axis=-1, keepdims=True)
    quat_n = (quaternions / (qnorm + 1e-8)).transpose(2, 0, 1).reshape(64, _CIN)
    ab = jnp.stack([alpha, beta], axis=1)
    out2d = _routing_call(agg_nm, quat_n, ab)
    return out2d.reshape(_N, _COUT, 4)


# speedup vs baseline: 2.2728x; 1.0102x over previous
import functools

import jax
import jax.numpy as jnp
from jax import lax
from jax.experimental import pallas as pl
from jax.experimental.pallas import tpu as pltpu
from jax.experimental.pallas import tpu_sc as plsc

_N = 50000
_E = 1600000
_CIN = 8
_COUT = 16
_F = _CIN * 4
_FH = _F // 2

_NC = 2
_NS = 16
_IW = 125
_CH_ROWS = 16
_CH = _IW * _CH_ROWS
_EPT = _E // _NS
_NCHUNK = _EPT // _CH
_NIO = 10
_RPT = _N // _NIO


@functools.lru_cache(maxsize=None)
def _make_sc_agg():
    mesh = plsc.VectorSubcoreMesh(core_axis_name="c", subcore_axis_name="s")

    @functools.partial(
        pl.kernel,
        mesh=mesh,
        compiler_params=pltpu.CompilerParams(use_tc_tiling_on_sc=False),
        out_type=jax.ShapeDtypeStruct((_N, _F), jnp.float32),
        scratch_types=[
            pltpu.VMEM((2, _CH_ROWS, _IW), jnp.int32),
            pltpu.VMEM((2, _CH_ROWS, _IW), jnp.int32),
            pltpu.VMEM((2, _CH, _FH), jnp.float32),
            pltpu.VMEM_SHARED((_N, _FH), jnp.float32),
            pltpu.SemaphoreType.DMA,
            pltpu.SemaphoreType.DMA,
            pltpu.SemaphoreType.DMA,
        ],
    )
    def _sc_agg(xs_hbm, src_hbm, dst_hbm, zero_hbm, out_hbm,
                gid_v, dst_v, rows_v, agg_sh, semg, semi, sems):
        c = lax.axis_index("c")
        s = lax.axis_index("s")

        @pl.when(s < _NIO)
        def _init():
            pltpu.sync_copy(zero_hbm, agg_sh.at[pl.ds(s * _RPT, _RPT)])
        plsc.subcore_barrier()

        row0 = s * (_EPT // _IW)

        def load_idx(k, b):
            base = row0 + k * _CH_ROWS
            pltpu.async_copy(src_hbm.at[pl.ds(base, _CH_ROWS)],
                             gid_v.at[b], semi)
            pltpu.async_copy(dst_hbm.at[pl.ds(base, _CH_ROWS)],
                             dst_v.at[b], semi)

        def wait_idx(b):
            pltpu.make_async_copy(src_hbm.at[pl.ds(0, _CH_ROWS)],
                                  gid_v.at[b], semi).wait()
            pltpu.make_async_copy(dst_hbm.at[pl.ds(0, _CH_ROWS)],
                                  dst_v.at[b], semi).wait()

        def fire_gathers(b):
            for j in range(_CH_ROWS):
                pltpu.async_copy(xs_hbm.at[c].at[gid_v.at[b, j]],
                                 rows_v.at[b, pl.ds(j * _IW, _IW)], semg)

        def wait_gathers(b):
            for j in range(_CH_ROWS):
                pltpu.make_async_copy(
                    xs_hbm.at[c].at[gid_v.at[b, j]],
                    rows_v.at[b, pl.ds(j * _IW, _IW)], semg).wait()

        def scatter_chunk(b):
            for j in range(_CH_ROWS):
                pltpu.async_copy(rows_v.at[b, pl.ds(j * _IW, _IW)],
                                 agg_sh.at[dst_v.at[b, j]], sems, add=True)
            for j in range(_CH_ROWS):
                pltpu.make_async_copy(rows_v.at[b, pl.ds(j * _IW, _IW)],
                                      agg_sh.at[dst_v.at[b, j]],
                                      sems).wait()

        load_idx(0, 0)
        wait_idx(0)
        fire_gathers(0)
        load_idx(1, 1)

        @pl.loop(0, _NCHUNK // 2)
        def _pair(m):
            for b in (0, 1):
                k = m * 2 + b

                @pl.when(k + 1 < _NCHUNK)
                def _prefetch():
                    wait_idx(1 - b)
                    fire_gathers(1 - b)

                wait_gathers(b)
                scatter_chunk(b)

                @pl.when(k + 2 < _NCHUNK)
                def _nextidx():
                    load_idx(k + 2, b)

        plsc.subcore_barrier()

        @pl.when(s < _NIO)
        def _writeback():
            pltpu.sync_copy(agg_sh.at[pl.ds(s * _RPT, _RPT)],
                            out_hbm.at[pl.ds(s * _RPT, _RPT),
                                       pl.ds(c * _FH, _FH)])

    return _sc_agg


_NB = 512


def _routing_body(agg_ref, quat_ref, ab_ref, out_ref):
    eps = 1e-8
    a = jnp.transpose(agg_ref[...])
    a = a.reshape(_CIN, 4, _NB)
    pw, px, py, pz = a[:, 0], a[:, 1], a[:, 2], a[:, 3]
    inv = 1.0 / (jnp.sqrt(pw * pw + px * px + py * py + pz * pz) + eps)
    pw, px, py, pz = pw * inv, px * inv, py * inv, pz * inv

    qw = quat_ref[0:16, :][:, :, None]
    qx = quat_ref[16:32, :][:, :, None]
    qy = quat_ref[32:48, :][:, :, None]
    qz = quat_ref[48:64, :][:, :, None]
    bw, bx, by, bz = pw[None], px[None], py[None], pz[None]
    vw = qw * bw - qx * bx - qy * by - qz * bz
    vx = qw * bx + qx * bw + qy * bz - qz * by
    vy = qw * by - qx * bz + qy * bw + qz * bx
    vz = qw * bz + qx * by - qy * bx + qz * bw

    def pose_and_agree(sw, sx, sy, sz):
        inv = 1.0 / (jnp.sqrt(sw * sw + sx * sx + sy * sy + sz * sz) + eps)
        ow, ox, oy, oz = sw * inv, sx * inv, sy * inv, sz * inv
        agree = (vw * ow[:, None] + vx * ox[:, None]
                 + vy * oy[:, None] + vz * oz[:, None])
        return ow, ox, oy, oz, agree

    sixteenth = jnp.float32(1.0 / _COUT)
    ow, ox, oy, oz, agree = pose_and_agree(
        vw.sum(axis=1) * sixteenth, vx.sum(axis=1) * sixteenth,
        vy.sum(axis=1) * sixteenth, vz.sum(axis=1) * sixteenth)
    b = agree

    for it in range(2):
        e = jnp.exp(b)
        cz = (1.0 / e.sum(axis=0))[None]
        c = e * cz
        ow, ox, oy, oz, agree = pose_and_agree(
            (c * vw).sum(axis=1), (c * vx).sum(axis=1),
            (c * vy).sum(axis=1), (c * vz).sum(axis=1))
        if it == 0:
            b = b + agree

    al = ab_ref[:, 0:1]
    be = ab_ref[:, 1:2]
    act = jax.nn.sigmoid(al * (agree.sum(axis=1) * jnp.float32(0.125)) + be)
    out = jnp.stack([ow * act, ox * act, oy * act, oz * act], axis=1)
    out_ref[...] = jnp.transpose(out.reshape(64, _NB))


def _routing_call(agg_nm, quat_n, ab):
    grid = (pl.cdiv(_N, _NB),)
    return pl.pallas_call(
        _routing_body,
        grid=grid,
        in_specs=[
            pl.BlockSpec((_NB, _F), lambda i: (i, 0)),
            pl.BlockSpec((64, 8), lambda i: (0, 0)),
            pl.BlockSpec((16, 2), lambda i: (0, 0)),
        ],
        out_specs=pl.BlockSpec((_NB, 64), lambda i: (i, 0)),
        out_shape=jax.ShapeDtypeStruct((_N, 64), jnp.float32),
    )(agg_nm, quat_n, ab)


def kernel(x, edge_index, quaternions, alpha, beta):
    xs = x.reshape(_N, 2, _FH).transpose(1, 0, 2)
    src2 = edge_index[0].reshape(_E // _IW, _IW)
    dst2 = edge_index[1].reshape(_E // _IW, _IW)
    zero = jnp.zeros((_RPT, _FH), jnp.float32)
    agg_nm = _make_sc_agg()(xs, src2, dst2, zero)
    qnorm = jnp.linalg.norm(quaternions, axis=-1, keepdims=True)
    quat_n = (quaternions / (qnorm + 1e-8)).transpose(2, 0, 1).reshape(64, _CIN)
    ab = jnp.stack([alpha, beta], axis=1)
    out2d = _routing_call(agg_nm, quat_n, ab)
    return out2d.reshape(_N, _COUT, 4)


# speedup vs baseline: 2.4804x; 1.0914x over previous
import functools

import jax
import jax.numpy as jnp
from jax import lax
from jax.experimental import pallas as pl
from jax.experimental.pallas import tpu as pltpu
from jax.experimental.pallas import tpu_sc as plsc

_N = 50000
_E = 1600000
_CIN = 8
_COUT = 16
_F = _CIN * 4
_FH = _F // 2

_NC = 2
_NS = 16
_IW = 125
_CH_ROWS = 16
_CH = _IW * _CH_ROWS
_EPT = _E // _NS
_NCHUNK = _EPT // _CH
_NIO = 10
_RPT = _N // _NIO


@functools.lru_cache(maxsize=None)
def _make_sc_agg():
    mesh = plsc.VectorSubcoreMesh(core_axis_name="c", subcore_axis_name="s")

    @functools.partial(
        pl.kernel,
        mesh=mesh,
        compiler_params=pltpu.CompilerParams(use_tc_tiling_on_sc=False),
        out_type=jax.ShapeDtypeStruct((_N, _F), jnp.float32),
        scratch_types=[
            pltpu.VMEM((2, _CH_ROWS, _IW), jnp.int32),
            pltpu.VMEM((2, _CH_ROWS, _IW), jnp.int32),
            pltpu.VMEM((2, _CH, _FH), jnp.float32),
            pltpu.VMEM_SHARED((_N, _FH), jnp.float32),
            pltpu.SemaphoreType.DMA,
            pltpu.SemaphoreType.DMA,
            pltpu.SemaphoreType.DMA,
        ],
    )
    def _sc_agg(xs_hbm, src_hbm, dst_hbm, zero_hbm, out_hbm,
                gid_v, dst_v, rows_v, agg_sh, semg, semi, sems):
        c = lax.axis_index("c")
        s = lax.axis_index("s")

        @pl.when(s < _NIO)
        def _init():
            pltpu.sync_copy(zero_hbm, agg_sh.at[pl.ds(s * _RPT, _RPT)])
        plsc.subcore_barrier()

        row0 = s * (_EPT // _IW)

        def load_idx(k, b):
            base = row0 + k * _CH_ROWS
            pltpu.async_copy(src_hbm.at[pl.ds(base, _CH_ROWS)],
                             gid_v.at[b], semi)
            pltpu.async_copy(dst_hbm.at[pl.ds(base, _CH_ROWS)],
                             dst_v.at[b], semi)

        def wait_idx(b):
            pltpu.make_async_copy(src_hbm.at[pl.ds(0, _CH_ROWS)],
                                  gid_v.at[b], semi).wait()
            pltpu.make_async_copy(dst_hbm.at[pl.ds(0, _CH_ROWS)],
                                  dst_v.at[b], semi).wait()

        def fire_gathers(b):
            for j in range(_CH_ROWS):
                pltpu.async_copy(xs_hbm.at[c].at[gid_v.at[b, j]],
                                 rows_v.at[b, pl.ds(j * _IW, _IW)], semg)

        def wait_gathers(b):
            for j in range(_CH_ROWS):
                pltpu.make_async_copy(
                    xs_hbm.at[c].at[gid_v.at[b, j]],
                    rows_v.at[b, pl.ds(j * _IW, _IW)], semg).wait()

        def scatter_chunk(b):
            for j in range(_CH_ROWS):
                pltpu.async_copy(rows_v.at[b, pl.ds(j * _IW, _IW)],
                                 agg_sh.at[dst_v.at[b, j]], sems, add=True)
            for j in range(_CH_ROWS):
                pltpu.make_async_copy(rows_v.at[b, pl.ds(j * _IW, _IW)],
                                      agg_sh.at[dst_v.at[b, j]],
                                      sems).wait()

        load_idx(0, 0)
        wait_idx(0)
        fire_gathers(0)
        load_idx(1, 1)

        @pl.loop(0, _NCHUNK // 2)
        def _pair(m):
            for b in (0, 1):
                k = m * 2 + b

                @pl.when(k + 1 < _NCHUNK)
                def _prefetch():
                    wait_idx(1 - b)
                    fire_gathers(1 - b)

                wait_gathers(b)
                scatter_chunk(b)

                @pl.when(k + 2 < _NCHUNK)
                def _nextidx():
                    load_idx(k + 2, b)

        plsc.subcore_barrier()

        @pl.when(s < _NIO)
        def _writeback():
            pltpu.sync_copy(agg_sh.at[pl.ds(s * _RPT, _RPT)],
                            out_hbm.at[pl.ds(s * _RPT, _RPT),
                                       pl.ds(c * _FH, _FH)])

    return _sc_agg


_NB = 1024


def _routing_body(agg_ref, quat_ref, ab_ref, out_ref):
    eps = 1e-8
    a = jnp.transpose(agg_ref[...])
    a = a.reshape(_CIN, 4, _NB)
    pw, px, py, pz = a[:, 0], a[:, 1], a[:, 2], a[:, 3]
    inv = 1.0 / (jnp.sqrt(pw * pw + px * px + py * py + pz * pz) + eps)
    pw, px, py, pz = pw * inv, px * inv, py * inv, pz * inv

    qw = quat_ref[0:16, :][:, :, None]
    qx = quat_ref[16:32, :][:, :, None]
    qy = quat_ref[32:48, :][:, :, None]
    qz = quat_ref[48:64, :][:, :, None]
    bw, bx, by, bz = pw[None], px[None], py[None], pz[None]
    vw = qw * bw - qx * bx - qy * by - qz * bz
    vx = qw * bx + qx * bw + qy * bz - qz * by
    vy = qw * by - qx * bz + qy * bw + qz * bx
    vz = qw * bz + qx * by - qy * bx + qz * bw

    def pose_and_agree(sw, sx, sy, sz):
        inv = 1.0 / (jnp.sqrt(sw * sw + sx * sx + sy * sy + sz * sz) + eps)
        ow, ox, oy, oz = sw * inv, sx * inv, sy * inv, sz * inv
        agree = (vw * ow[:, None] + vx * ox[:, None]
                 + vy * oy[:, None] + vz * oz[:, None])
        return ow, ox, oy, oz, agree

    sixteenth = jnp.float32(1.0 / _COUT)
    ow, ox, oy, oz, agree = pose_and_agree(
        vw.sum(axis=1) * sixteenth, vx.sum(axis=1) * sixteenth,
        vy.sum(axis=1) * sixteenth, vz.sum(axis=1) * sixteenth)
    b = agree

    for it in range(2):
        e = jnp.exp(b)
        cz = (1.0 / e.sum(axis=0))[None]
        c = e * cz
        ow, ox, oy, oz, agree = pose_and_agree(
            (c * vw).sum(axis=1), (c * vx).sum(axis=1),
            (c * vy).sum(axis=1), (c * vz).sum(axis=1))
        if it == 0:
            b = b + agree

    al = ab_ref[:, 0:1]
    be = ab_ref[:, 1:2]
    act = jax.nn.sigmoid(al * (agree.sum(axis=1) * jnp.float32(0.125)) + be)
    out = jnp.stack([ow * act, ox * act, oy * act, oz * act], axis=1)
    out_ref[...] = jnp.transpose(out.reshape(64, _NB))


def _routing_call(agg_nm, quat_n, ab):
    grid = (pl.cdiv(_N, _NB),)
    return pl.pallas_call(
        _routing_body,
        grid=grid,
        in_specs=[
            pl.BlockSpec((_NB, _F), lambda i: (i, 0)),
            pl.BlockSpec((64, 8), lambda i: (0, 0)),
            pl.BlockSpec((16, 2), lambda i: (0, 0)),
        ],
        out_specs=pl.BlockSpec((_NB, 64), lambda i: (i, 0)),
        out_shape=jax.ShapeDtypeStruct((_N, 64), jnp.float32),
    )(agg_nm, quat_n, ab)


def kernel(x, edge_index, quaternions, alpha, beta):
    xs = x.reshape(_N, 2, _FH).transpose(1, 0, 2)
    src2 = edge_index[0].reshape(_E // _IW, _IW)
    dst2 = edge_index[1].reshape(_E // _IW, _IW)
    zero = jnp.zeros((_RPT, _FH), jnp.float32)
    agg_nm = _make_sc_agg()(xs, src2, dst2, zero)
    qnorm = jnp.linalg.norm(quaternions, axis=-1, keepdims=True)
    quat_n = (quaternions / (qnorm + 1e-8)).transpose(2, 0, 1).reshape(64, _CIN)
    ab = jnp.stack([alpha, beta], axis=1)
    out2d = _routing_call(agg_nm, quat_n, ab)
    return out2d.reshape(_N, _COUT, 4)
